# baseline (speedup 1.0000x reference)
import jax
import jax.numpy as jnp
from jax.experimental import pallas as pl

R = 16


def _norm_body(h_ref, o_ref):
    h = h_ref[:]
    n = jnp.sqrt(jnp.sum(h * h, axis=1, keepdims=True))
    o_ref[:] = h / jnp.maximum(n, 1e-6)


def _layer(x, edge_index, edge_type, basis, comp, root, bias):
    n = x.shape[0]
    src = edge_index[0]
    dst = edge_index[1]
    seg = dst * R + edge_type
    msum = jax.ops.segment_sum(x[src], seg, num_segments=n * R)
    cnt = jax.ops.segment_sum(jnp.ones((src.shape[0],), jnp.float32), seg,
                              num_segments=n * R)
    mean = msum / jnp.maximum(cnt, 1.0)[:, None]
    mean = mean.reshape(n, R, x.shape[1])
    W = jnp.einsum('rb,bio->rio', comp, basis)
    out = jnp.einsum('nri,rio->no', mean, W)
    return out + x @ root + bias


def kernel(x, edge_index, edge_type, basis1, comp1, root1, bias1,
           basis2, comp2, root2, bias2):
    h = _layer(x, edge_index, edge_type, basis1, comp1, root1, bias1)
    h = jax.nn.relu(h)
    h = _layer(h, edge_index, edge_type, basis2, comp2, root2, bias2)
    return pl.pallas_call(
        _norm_body,
        out_shape=jax.ShapeDtypeStruct(h.shape, h.dtype),
    )(h)


# speedup vs baseline: 3.2748x; 3.2748x over previous
import functools

import jax
import jax.numpy as jnp
from jax import lax
from jax.experimental import pallas as pl
from jax.experimental.pallas import tpu as pltpu
from jax.experimental.pallas import tpu_sc as plsc

N = 10000
E = 320000
R = 16
NB = 8
D_IN = 128
DH = 64
DO = 32
NR = N * R

NC = 2
NS = 16
NW = NC * NS
CK = 128
C = E // CK
NPAD = 10240
NSL = NPAD // NS
RSL = NR // NS

_f32 = jnp.float32
_i32 = jnp.int32


def _prep_body(e2_ref, et_ref, g_ref, s_ref, d_ref):
    src = e2_ref[0]
    dst = e2_ref[1]
    t = et_ref[:]
    g_ref[:] = src * R + t
    s_ref[:] = dst * R + t
    d_ref[:] = dst


def _invrep_body(cnt_ref, inv_ref):
    c = cnt_ref[0] + cnt_ref[1]
    inv = 1.0 / jnp.maximum(c, 1.0)
    inv_ref[:] = jnp.broadcast_to(inv, (inv.shape[0], 16))


def _make_lin_body(d_in, d_out):
    def body(h_ref, bf_ref, comp_ref, root_ref, bias_ref, u_ref, xr_ref):
        h = h_ref[:]
        z = jnp.dot(h, bf_ref[:], preferred_element_type=_f32)
        comp = comp_ref[:]
        for r in range(R):
            acc = z[:, 0:d_out] * comp[r:r + 1, 0:1]
            for b in range(1, NB):
                acc = acc + z[:, b * d_out:(b + 1) * d_out] * comp[r:r + 1, b:b + 1]
            u_ref[:, r * d_out:(r + 1) * d_out] = acc
        xr_ref[:] = jnp.dot(h, root_ref[:], preferred_element_type=_f32) + bias_ref[:]
    return body


def _mid_body(ap_ref, xr1_ref, bf_ref, comp_ref, root_ref, bias_ref, u_ref, xr_ref):
    h = jnp.maximum(ap_ref[0] + ap_ref[1] + xr1_ref[:], 0.0)
    z = jnp.dot(h, bf_ref[:], preferred_element_type=_f32)
    comp = comp_ref[:]
    for r in range(R):
        acc = z[:, 0:DO] * comp[r:r + 1, 0:1]
        for b in range(1, NB):
            acc = acc + z[:, b * DO:(b + 1) * DO] * comp[r:r + 1, b:b + 1]
        u_ref[:, r * DO:(r + 1) * DO] = acc
    xr_ref[:] = jnp.dot(h, root_ref[:], preferred_element_type=_f32) + bias_ref[:]


def _fin_body(ap_ref, xr2_ref, o_ref):
    o = ap_ref[0] + ap_ref[1] + xr2_ref[:]
    nrm = jnp.sqrt(jnp.sum(o * o, axis=1, keepdims=True))
    o_ref[:] = o / jnp.maximum(nrm, 1e-6)


_BN = 1000


def _tc_lin1(x, bf1, comp1, root1, bias1):
    return pl.pallas_call(
        _make_lin_body(D_IN, DH),
        grid=(N // _BN,),
        in_specs=[
            pl.BlockSpec((_BN, D_IN), lambda i: (i, 0)),
            pl.BlockSpec((D_IN, NB * DH), lambda i: (0, 0)),
            pl.BlockSpec((R, NB), lambda i: (0, 0)),
            pl.BlockSpec((D_IN, DH), lambda i: (0, 0)),
            pl.BlockSpec((1, DH), lambda i: (0, 0)),
        ],
        out_specs=[
            pl.BlockSpec((_BN, R * DH), lambda i: (i, 0)),
            pl.BlockSpec((_BN, DH), lambda i: (i, 0)),
        ],
        out_shape=[
            jax.ShapeDtypeStruct((N, R * DH), _f32),
            jax.ShapeDtypeStruct((N, DH), _f32),
        ],
    )(x, bf1, comp1, root1, bias1)


def _tc_mid(accp1, xr1, bf2, comp2, root2, bias2):
    return pl.pallas_call(
        _mid_body,
        grid=(N // _BN,),
        in_specs=[
            pl.BlockSpec((NC, _BN, DH), lambda i: (0, i, 0)),
            pl.BlockSpec((_BN, DH), lambda i: (i, 0)),
            pl.BlockSpec((DH, NB * DO), lambda i: (0, 0)),
            pl.BlockSpec((R, NB), lambda i: (0, 0)),
            pl.BlockSpec((DH, DO), lambda i: (0, 0)),
            pl.BlockSpec((1, DO), lambda i: (0, 0)),
        ],
        out_specs=[
            pl.BlockSpec((_BN, R * DO), lambda i: (i, 0)),
            pl.BlockSpec((_BN, DO), lambda i: (i, 0)),
        ],
        out_shape=[
            jax.ShapeDtypeStruct((N, R * DO), _f32),
            jax.ShapeDtypeStruct((N, DO), _f32),
        ],
    )(accp1, xr1, bf2, comp2, root2, bias2)


def _tc_fin(accp2, xr2):
    return pl.pallas_call(
        _fin_body,
        grid=(N // _BN,),
        in_specs=[
            pl.BlockSpec((NC, _BN, DO), lambda i: (0, i, 0)),
            pl.BlockSpec((_BN, DO), lambda i: (i, 0)),
        ],
        out_specs=pl.BlockSpec((_BN, DO), lambda i: (i, 0)),
        out_shape=jax.ShapeDtypeStruct((N, DO), _f32),
    )(accp2, xr2)


def _sc_mesh():
    return plsc.VectorSubcoreMesh(core_axis_name="c", subcore_axis_name="s")


def _count_body(s_hbm, cntp, cnt_sh, sidx, ones, zb, sem):
    cc = lax.axis_index("c")
    ss = lax.axis_index("s")
    w = cc * NS + ss

    @pl.loop(0, RSL // 16)
    def _zb(i):
        zb[pl.ds(i * 16, 16)] = jnp.zeros((16,), _f32)

    @pl.loop(0, CK // 16)
    def _ones(i):
        ones[pl.ds(i * 16, 16)] = jnp.full((16,), 1.0, _f32)

    pltpu.sync_copy(zb, cnt_sh.at[pl.ds(ss * RSL, RSL)])
    plsc.subcore_barrier()

    @pl.loop(w, C, step=NW)
    def _chunk(j):
        pltpu.async_copy(s_hbm.at[j], sidx, sem).wait()
        pltpu.sync_copy(ones, cnt_sh.at[sidx], add=True)

    plsc.subcore_barrier()
    pltpu.sync_copy(cnt_sh.at[pl.ds(ss * RSL, RSL)], zb)
    pltpu.sync_copy(zb, cntp.at[pl.ds(cc * NR + ss * RSL, RSL)])


def _sc_count(sarr):
    kern = functools.partial(
        pl.kernel,
        out_type=jax.ShapeDtypeStruct((NC * NR,), _f32),
        mesh=_sc_mesh(),
        compiler_params=pltpu.CompilerParams(use_tc_tiling_on_sc=False),
        scratch_types=[
            pltpu.VMEM_SHARED((NR,), _f32),
            pltpu.VMEM((CK,), _i32),
            pltpu.VMEM((CK,), _f32),
            pltpu.VMEM((RSL,), _f32),
            pltpu.SemaphoreType.DMA,
        ],
    )(_count_body)
    return kern(sarr)


def _make_edge_body(dh):
    def body(u_hbm, inv_hbm, g_hbm, s_hbm, d_hbm, accp,
             acc_sh, gi, si, di, sc, rows, zrow, s1, s2, s3):
        cc = lax.axis_index("c")
        ss = lax.axis_index("s")
        w = cc * NS + ss

        @pl.loop(0, NSL)
        def _zero(i):
            for q in range(dh // 16):
                zrow[i, pl.ds(q * 16, 16)] = jnp.zeros((16,), _f32)

        pltpu.sync_copy(zrow, acc_sh.at[pl.ds(ss * NSL, NSL)])
        plsc.subcore_barrier()

        @pl.loop(w, C, step=NW)
        def _chunk(j):
            c1 = pltpu.async_copy(g_hbm.at[j], gi, s1)
            c2 = pltpu.async_copy(s_hbm.at[j], si, s2)
            c3 = pltpu.async_copy(d_hbm.at[j], di, s3)
            c1.wait()
            c2.wait()
            c3.wait()
            c4 = pltpu.async_copy(inv_hbm.at[si], sc, s1)
            c5 = pltpu.async_copy(u_hbm.at[gi], rows, s2)
            c4.wait()
            c5.wait()

            @pl.loop(0, CK)
            def _scale(e):
                sv = sc[e, pl.ds(0, 16)]
                for q in range(dh // 16):
                    v = rows[e, pl.ds(q * 16, 16)]
                    rows[e, pl.ds(q * 16, 16)] = v * sv

            pltpu.sync_copy(rows, acc_sh.at[di], add=True)

        plsc.subcore_barrier()
        pltpu.sync_copy(acc_sh.at[pl.ds(ss * NSL, NSL)], zrow)
        pltpu.sync_copy(zrow, accp.at[pl.ds(cc * NPAD + ss * NSL, NSL)])
    return body


def _sc_edge_pass(u, inv, garr, sarr, darr, dh):
    kern = functools.partial(
        pl.kernel,
        out_type=jax.ShapeDtypeStruct((NC * NPAD, dh), _f32),
        mesh=_sc_mesh(),
        compiler_params=pltpu.CompilerParams(use_tc_tiling_on_sc=False),
        scratch_types=[
            pltpu.VMEM_SHARED((NPAD, dh), _f32),
            pltpu.VMEM((CK,), _i32),
            pltpu.VMEM((CK,), _i32),
            pltpu.VMEM((CK,), _i32),
            pltpu.VMEM((CK, 16), _f32),
            pltpu.VMEM((CK, dh), _f32),
            pltpu.VMEM((NSL, dh), _f32),
            pltpu.SemaphoreType.DMA,
            pltpu.SemaphoreType.DMA,
            pltpu.SemaphoreType.DMA,
        ],
    )(_make_edge_body(dh))
    return kern(u, inv, garr, sarr, darr)


def kernel(x, edge_index, edge_type, basis1, comp1, root1, bias1,
           basis2, comp2, root2, bias2):
    e2 = edge_index.reshape(2, C, CK)
    et2 = edge_type.reshape(C, CK)

    garr, sarr, darr = pl.pallas_call(
        _prep_body,
        out_shape=[
            jax.ShapeDtypeStruct((C, CK), _i32),
            jax.ShapeDtypeStruct((C, CK), _i32),
            jax.ShapeDtypeStruct((C, CK), _i32),
        ],
    )(e2, et2)

    cntp = _sc_count(sarr)

    invrep = pl.pallas_call(
        _invrep_body,
        grid=(NR // 2000,),
        in_specs=[pl.BlockSpec((NC, 2000, 1), lambda i: (0, i, 0))],
        out_specs=pl.BlockSpec((2000, 16), lambda i: (i, 0)),
        out_shape=jax.ShapeDtypeStruct((NR, 16), _f32),
    )(cntp.reshape(NC, NR, 1))

    bf1 = jnp.transpose(basis1, (1, 0, 2)).reshape(D_IN, NB * DH)
    u1, xr1 = _tc_lin1(x, bf1, comp1, root1, bias1.reshape(1, DH))

    accp1 = _sc_edge_pass(u1.reshape(NR, DH), invrep, garr, sarr, darr, DH)

    bf2 = jnp.transpose(basis2, (1, 0, 2)).reshape(DH, NB * DO)
    u2, xr2 = _tc_mid(accp1.reshape(NC, NPAD, DH)[:, :N], xr1, bf2, comp2, root2,
                      bias2.reshape(1, DO))

    accp2 = _sc_edge_pass(u2.reshape(NR, DO), invrep, garr, sarr, darr, DO)

    return _tc_fin(accp2.reshape(NC, NPAD, DO)[:, :N], xr2)


# speedup vs baseline: 3.5271x; 1.0771x over previous
import functools

import jax
import jax.numpy as jnp
from jax import lax
from jax.experimental import pallas as pl
from jax.experimental.pallas import tpu as pltpu
from jax.experimental.pallas import tpu_sc as plsc

N = 10000
E = 320000
R = 16
NB = 8
D_IN = 128
DH = 64
DO = 32
NR = N * R

NC = 2
NS = 16
NW = NC * NS
CK = 80
C = E // CK
CPW = 126
PADC = NW * CPW
PADSEG = NR
NSL = N // NS
RSL = NR // NS

_f32 = jnp.float32
_i32 = jnp.int32


def _prep_body(e2_ref, et_ref, g_ref, s_ref, d_ref):
    src = e2_ref[0]
    dst = e2_ref[1]
    t = et_ref[:]
    g_ref[:] = src * R + t
    s_ref[:] = dst * R + t
    d_ref[:] = dst


def _invrep_body(cnt_ref, inv_ref):
    c = cnt_ref[0] + cnt_ref[1]
    inv = 1.0 / jnp.maximum(c, 1.0)
    inv_ref[:] = jnp.broadcast_to(inv, (inv.shape[0], 16))


def _make_lin_body(d_in, d_out):
    def body(h_ref, bf_ref, comp_ref, root_ref, bias_ref, u_ref, xr_ref):
        h = h_ref[:]
        z = jnp.dot(h, bf_ref[:], preferred_element_type=_f32)
        comp = comp_ref[:]
        for r in range(R):
            acc = z[:, 0:d_out] * comp[r:r + 1, 0:1]
            for b in range(1, NB):
                acc = acc + z[:, b * d_out:(b + 1) * d_out] * comp[r:r + 1, b:b + 1]
            u_ref[:, r * d_out:(r + 1) * d_out] = acc
        xr_ref[:] = jnp.dot(h, root_ref[:], preferred_element_type=_f32) + bias_ref[:]
    return body


def _mid_body(ap_ref, xr1_ref, bf_ref, comp_ref, root_ref, bias_ref, u_ref, xr_ref):
    h = jnp.maximum(ap_ref[0] + ap_ref[1] + xr1_ref[:], 0.0)
    z = jnp.dot(h, bf_ref[:], preferred_element_type=_f32)
    comp = comp_ref[:]
    for r in range(R):
        acc = z[:, 0:DO] * comp[r:r + 1, 0:1]
        for b in range(1, NB):
            acc = acc + z[:, b * DO:(b + 1) * DO] * comp[r:r + 1, b:b + 1]
        u_ref[:, r * DO:(r + 1) * DO] = acc
    xr_ref[:] = jnp.dot(h, root_ref[:], preferred_element_type=_f32) + bias_ref[:]


def _fin_body(ap_ref, xr2_ref, o_ref):
    o = ap_ref[0] + ap_ref[1] + xr2_ref[:]
    nrm = jnp.sqrt(jnp.sum(o * o, axis=1, keepdims=True))
    o_ref[:] = o / jnp.maximum(nrm, 1e-6)


_BN = 1000


def _tc_lin1(x, bf1, comp1, root1, bias1):
    return pl.pallas_call(
        _make_lin_body(D_IN, DH),
        grid=(N // _BN,),
        in_specs=[
            pl.BlockSpec((_BN, D_IN), lambda i: (i, 0)),
            pl.BlockSpec((D_IN, NB * DH), lambda i: (0, 0)),
            pl.BlockSpec((R, NB), lambda i: (0, 0)),
            pl.BlockSpec((D_IN, DH), lambda i: (0, 0)),
            pl.BlockSpec((1, DH), lambda i: (0, 0)),
        ],
        out_specs=[
            pl.BlockSpec((_BN, R * DH), lambda i: (i, 0)),
            pl.BlockSpec((_BN, DH), lambda i: (i, 0)),
        ],
        out_shape=[
            jax.ShapeDtypeStruct((N, R * DH), _f32),
            jax.ShapeDtypeStruct((N, DH), _f32),
        ],
    )(x, bf1, comp1, root1, bias1)


def _tc_mid(accp1, xr1, bf2, comp2, root2, bias2):
    return pl.pallas_call(
        _mid_body,
        grid=(N // _BN,),
        in_specs=[
            pl.BlockSpec((NC, _BN, DH), lambda i: (0, i, 0)),
            pl.BlockSpec((_BN, DH), lambda i: (i, 0)),
            pl.BlockSpec((DH, NB * DO), lambda i: (0, 0)),
            pl.BlockSpec((R, NB), lambda i: (0, 0)),
            pl.BlockSpec((DH, DO), lambda i: (0, 0)),
            pl.BlockSpec((1, DO), lambda i: (0, 0)),
        ],
        out_specs=[
            pl.BlockSpec((_BN, R * DO), lambda i: (i, 0)),
            pl.BlockSpec((_BN, DO), lambda i: (i, 0)),
        ],
        out_shape=[
            jax.ShapeDtypeStruct((N, R * DO), _f32),
            jax.ShapeDtypeStruct((N, DO), _f32),
        ],
    )(accp1, xr1, bf2, comp2, root2, bias2)


def _tc_fin(accp2, xr2):
    return pl.pallas_call(
        _fin_body,
        grid=(N // _BN,),
        in_specs=[
            pl.BlockSpec((NC, _BN, DO), lambda i: (0, i, 0)),
            pl.BlockSpec((_BN, DO), lambda i: (i, 0)),
        ],
        out_specs=pl.BlockSpec((_BN, DO), lambda i: (i, 0)),
        out_shape=jax.ShapeDtypeStruct((N, DO), _f32),
    )(accp2, xr2)


def _sc_mesh():
    return plsc.VectorSubcoreMesh(core_axis_name="c", subcore_axis_name="s")


def _count_body(s_hbm, cntp, cnt_sh, sidx, ones, zb, sem):
    cc = lax.axis_index("c")
    ss = lax.axis_index("s")
    w = cc * NS + ss

    @pl.loop(0, RSL // 16)
    def _zb(i):
        zb[pl.ds(i * 16, 16)] = jnp.zeros((16,), _f32)

    @pl.loop(0, CK // 16)
    def _ones(i):
        ones[pl.ds(i * 16, 16)] = jnp.full((16,), 1.0, _f32)

    pltpu.sync_copy(zb, cnt_sh.at[pl.ds(ss * RSL, RSL)])
    plsc.subcore_barrier()

    @pl.loop(w, C, step=NW)
    def _chunk(j):
        pltpu.async_copy(s_hbm.at[j], sidx, sem).wait()
        pltpu.sync_copy(ones, cnt_sh.at[sidx], add=True)

    plsc.subcore_barrier()
    pltpu.sync_copy(cnt_sh.at[pl.ds(ss * RSL, RSL)], zb)
    pltpu.sync_copy(zb, cntp.at[pl.ds(cc * NR + ss * RSL, RSL)])


def _sc_count(sarr):
    kern = functools.partial(
        pl.kernel,
        out_type=jax.ShapeDtypeStruct((NC * NR,), _f32),
        mesh=_sc_mesh(),
        compiler_params=pltpu.CompilerParams(use_tc_tiling_on_sc=False),
        scratch_types=[
            pltpu.VMEM_SHARED((NR,), _f32),
            pltpu.VMEM((CK,), _i32),
            pltpu.VMEM((CK,), _f32),
            pltpu.VMEM((RSL,), _f32),
            pltpu.SemaphoreType.DMA,
        ],
    )(_count_body)
    return kern(sarr)


def _make_edge_body(dh):
    def body(u_hbm, inv_hbm, g_hbm, s_hbm, d_hbm, accp,
             acc_sh, giA, siA, diA, scA, scB, rowsA, rowsB, zrow,
             su_a, su_b, ss_a, ss_b):
        cc = lax.axis_index("c")
        ss = lax.axis_index("s")
        w = cc * NS + ss

        @pl.loop(0, NSL)
        def _zero(i):
            for q in range(dh // 16):
                zrow[i, pl.ds(q * 16, 16)] = jnp.zeros((16,), _f32)

        pltpu.sync_copy(zrow, acc_sh.at[pl.ds(ss * NSL, NSL)])

        pltpu.sync_copy(g_hbm.at[pl.ds(w * CPW, CPW)], giA)
        pltpu.sync_copy(s_hbm.at[pl.ds(w * CPW, CPW)], siA)
        pltpu.sync_copy(d_hbm.at[pl.ds(w * CPW, CPW)], diA)
        plsc.subcore_barrier()

        def issue(j, rows_buf, sc_buf, sem_u, sem_s):
            pltpu.async_copy(u_hbm.at[giA.at[j]], rows_buf, sem_u)
            pltpu.async_copy(inv_hbm.at[siA.at[j]], sc_buf, sem_s)

        def drain(rows_buf, sc_buf, sem_u, sem_s):
            pltpu.make_async_copy(u_hbm.at[giA.at[0]], rows_buf, sem_u).wait()
            pltpu.make_async_copy(inv_hbm.at[siA.at[0]], sc_buf, sem_s).wait()

        def process(j, rows_buf, sc_buf, sem_u, sem_s):
            drain(rows_buf, sc_buf, sem_u, sem_s)

            @pl.loop(0, CK, unroll=4)
            def _scale(e):
                sv = sc_buf[e, pl.ds(0, 16)]
                for q in range(dh // 16):
                    v = rows_buf[e, pl.ds(q * 16, 16)]
                    rows_buf[e, pl.ds(q * 16, 16)] = v * sv

            pltpu.sync_copy(rows_buf, acc_sh.at[diA.at[j]], add=True)

        issue(0, rowsA, scA, su_a, ss_a)

        @pl.loop(0, CPW // 2)
        def _pair(i):
            j0 = 2 * i
            j1 = 2 * i + 1
            issue(j1, rowsB, scB, su_b, ss_b)
            process(j0, rowsA, scA, su_a, ss_a)
            issue(jnp.minimum(j1 + 1, CPW - 1), rowsA, scA, su_a, ss_a)
            process(j1, rowsB, scB, su_b, ss_b)

        drain(rowsA, scA, su_a, ss_a)

        plsc.subcore_barrier()
        pltpu.sync_copy(acc_sh.at[pl.ds(ss * NSL, NSL)], zrow)
        pltpu.sync_copy(zrow, accp.at[pl.ds(cc * N + ss * NSL, NSL)])
    return body


def _sc_edge_pass(u, inv, garr, sarr, darr, dh):
    kern = functools.partial(
        pl.kernel,
        out_type=pltpu.HBM((NC * N, dh), _f32),
        mesh=_sc_mesh(),
        compiler_params=pltpu.CompilerParams(use_tc_tiling_on_sc=False),
        scratch_types=[
            pltpu.VMEM_SHARED((N, dh), _f32),
            pltpu.VMEM((CPW, CK), _i32),
            pltpu.VMEM((CPW, CK), _i32),
            pltpu.VMEM((CPW, CK), _i32),
            pltpu.VMEM((CK, 16), _f32),
            pltpu.VMEM((CK, 16), _f32),
            pltpu.VMEM((CK, dh), _f32),
            pltpu.VMEM((CK, dh), _f32),
            pltpu.VMEM((NSL, dh), _f32),
            pltpu.SemaphoreType.DMA,
            pltpu.SemaphoreType.DMA,
            pltpu.SemaphoreType.DMA,
            pltpu.SemaphoreType.DMA,
        ],
    )(_make_edge_body(dh))
    return kern(u, inv, garr, sarr, darr)


def kernel(x, edge_index, edge_type, basis1, comp1, root1, bias1,
           basis2, comp2, root2, bias2):
    e2 = edge_index.reshape(2, C, CK)
    et2 = edge_type.reshape(C, CK)

    garr, sarr, darr = pl.pallas_call(
        _prep_body,
        out_shape=[
            jax.ShapeDtypeStruct((C, CK), _i32),
            jax.ShapeDtypeStruct((C, CK), _i32),
            jax.ShapeDtypeStruct((C, CK), _i32),
        ],
    )(e2, et2)

    pad_i = jnp.zeros((PADC - C, CK), _i32)
    pad_s = jnp.full((PADC - C, CK), PADSEG, _i32)
    garr_p = jnp.concatenate([garr, pad_i], axis=0)
    sarr_p = jnp.concatenate([sarr, pad_s], axis=0)
    darr_p = jnp.concatenate([darr, pad_i], axis=0)

    cntp = _sc_count(sarr)

    invrep = pl.pallas_call(
        _invrep_body,
        grid=(NR // 2000,),
        in_specs=[pl.BlockSpec((NC, 2000, 1), lambda i: (0, i, 0))],
        out_specs=pl.BlockSpec((2000, 16), lambda i: (i, 0)),
        out_shape=jax.ShapeDtypeStruct((NR, 16), _f32),
    )(cntp.reshape(NC, NR, 1))
    invrep = jnp.concatenate([invrep, jnp.zeros((16, 16), _f32)], axis=0)

    bf1 = jnp.transpose(basis1, (1, 0, 2)).reshape(D_IN, NB * DH)
    u1, xr1 = _tc_lin1(x, bf1, comp1, root1, bias1.reshape(1, DH))

    accp1 = _sc_edge_pass(u1.reshape(NR, DH), invrep, garr_p, sarr_p, darr_p, DH)

    bf2 = jnp.transpose(basis2, (1, 0, 2)).reshape(DH, NB * DO)
    u2, xr2 = _tc_mid(accp1.reshape(NC, N, DH), xr1, bf2, comp2, root2,
                      bias2.reshape(1, DO))

    accp2 = _sc_edge_pass(u2.reshape(NR, DO), invrep, garr_p, sarr_p, darr_p, DO)

    return _tc_fin(accp2.reshape(NC, N, DO), xr2)


# speedup vs baseline: 5.0080x; 1.4198x over previous
import functools

import jax
import jax.numpy as jnp
from jax import lax
from jax.experimental import pallas as pl
from jax.experimental.pallas import tpu as pltpu
from jax.experimental.pallas import tpu_sc as plsc

N = 10000
E = 320000
R = 16
NB = 8
D_IN = 128
DH = 64
DO = 32
NR = N * R

NC = 2
NS = 16
NW = NC * NS
CK = 80
C = E // CK
CPW = 126
PADC = NW * CPW
PADSEG = NR
NSL = N // NS
RSL = NR // NS

_f32 = jnp.float32
_i32 = jnp.int32


def _prep_body(e2_ref, et_ref, g_ref, s_ref, d_ref):
    src = e2_ref[0]
    dst = e2_ref[1]
    t = et_ref[:]
    g_ref[:] = src * R + t
    s_ref[:] = dst * R + t
    d_ref[:] = dst


def _invrep_body(cnt_ref, inv_ref):
    c = cnt_ref[0] + cnt_ref[1]
    inv = 1.0 / jnp.maximum(c, 1.0)
    inv_ref[:] = jnp.broadcast_to(inv, (inv.shape[0], 16))


def _make_lin_body(d_in, d_out):
    def body(h_ref, bf_ref, comp_ref, root_ref, bias_ref, u_ref, xr_ref):
        h = h_ref[:]
        bf = bf_ref[:]
        comp = comp_ref[:]
        for r in range(R):
            w = bf[:, 0:d_out] * comp[r:r + 1, 0:1]
            for b in range(1, NB):
                w = w + bf[:, b * d_out:(b + 1) * d_out] * comp[r:r + 1, b:b + 1]
            u_ref[:, r * d_out:(r + 1) * d_out] = jnp.dot(
                h, w, preferred_element_type=_f32)
        xr_ref[:] = jnp.dot(h, root_ref[:], preferred_element_type=_f32) + bias_ref[:]
    return body


def _mid_body(ap_ref, xr1_ref, bf_ref, comp_ref, root_ref, bias_ref, u_ref, xr_ref):
    h = jnp.maximum(ap_ref[0] + ap_ref[1] + xr1_ref[:], 0.0)
    bf = bf_ref[:]
    comp = comp_ref[:]
    for r in range(R):
        w = bf[:, 0:DO] * comp[r:r + 1, 0:1]
        for b in range(1, NB):
            w = w + bf[:, b * DO:(b + 1) * DO] * comp[r:r + 1, b:b + 1]
        u_ref[:, r * DO:(r + 1) * DO] = jnp.dot(h, w, preferred_element_type=_f32)
    xr_ref[:] = jnp.dot(h, root_ref[:], preferred_element_type=_f32) + bias_ref[:]


def _fin_body(ap_ref, xr2_ref, o_ref):
    o = ap_ref[0] + ap_ref[1] + xr2_ref[:]
    nrm = jnp.sqrt(jnp.sum(o * o, axis=1, keepdims=True))
    o_ref[:] = o / jnp.maximum(nrm, 1e-6)


_BN = 1000


def _tc_lin1(x, bf1, comp1, root1, bias1):
    return pl.pallas_call(
        _make_lin_body(D_IN, DH),
        grid=(N // _BN,),
        in_specs=[
            pl.BlockSpec((_BN, D_IN), lambda i: (i, 0)),
            pl.BlockSpec((D_IN, NB * DH), lambda i: (0, 0)),
            pl.BlockSpec((R, NB), lambda i: (0, 0)),
            pl.BlockSpec((D_IN, DH), lambda i: (0, 0)),
            pl.BlockSpec((1, DH), lambda i: (0, 0)),
        ],
        out_specs=[
            pl.BlockSpec((_BN, R * DH), lambda i: (i, 0)),
            pl.BlockSpec((_BN, DH), lambda i: (i, 0)),
        ],
        out_shape=[
            jax.ShapeDtypeStruct((N, R * DH), _f32),
            jax.ShapeDtypeStruct((N, DH), _f32),
        ],
    )(x, bf1, comp1, root1, bias1)


def _tc_mid(accp1, xr1, bf2, comp2, root2, bias2):
    return pl.pallas_call(
        _mid_body,
        grid=(N // _BN,),
        in_specs=[
            pl.BlockSpec((NC, _BN, DH), lambda i: (0, i, 0)),
            pl.BlockSpec((_BN, DH), lambda i: (i, 0)),
            pl.BlockSpec((DH, NB * DO), lambda i: (0, 0)),
            pl.BlockSpec((R, NB), lambda i: (0, 0)),
            pl.BlockSpec((DH, DO), lambda i: (0, 0)),
            pl.BlockSpec((1, DO), lambda i: (0, 0)),
        ],
        out_specs=[
            pl.BlockSpec((_BN, R * DO), lambda i: (i, 0)),
            pl.BlockSpec((_BN, DO), lambda i: (i, 0)),
        ],
        out_shape=[
            jax.ShapeDtypeStruct((N, R * DO), _f32),
            jax.ShapeDtypeStruct((N, DO), _f32),
        ],
    )(accp1, xr1, bf2, comp2, root2, bias2)


def _tc_fin(accp2, xr2):
    return pl.pallas_call(
        _fin_body,
        grid=(N // _BN,),
        in_specs=[
            pl.BlockSpec((NC, _BN, DO), lambda i: (0, i, 0)),
            pl.BlockSpec((_BN, DO), lambda i: (i, 0)),
        ],
        out_specs=pl.BlockSpec((_BN, DO), lambda i: (i, 0)),
        out_shape=jax.ShapeDtypeStruct((N, DO), _f32),
    )(accp2, xr2)


def _sc_mesh():
    return plsc.VectorSubcoreMesh(core_axis_name="c", subcore_axis_name="s")


def _count_body(s_hbm, cntp, cnt_sh, sidx, ones, zb, sem):
    cc = lax.axis_index("c")
    ss = lax.axis_index("s")
    w = cc * NS + ss

    @pl.loop(0, RSL // 16)
    def _zb(i):
        zb[pl.ds(i * 16, 16)] = jnp.zeros((16,), _f32)

    @pl.loop(0, CK // 16)
    def _ones(i):
        ones[pl.ds(i * 16, 16)] = jnp.full((16,), 1.0, _f32)

    pltpu.sync_copy(zb, cnt_sh.at[pl.ds(ss * RSL, RSL)])
    plsc.subcore_barrier()

    @pl.loop(w, C, step=NW)
    def _chunk(j):
        pltpu.async_copy(s_hbm.at[j], sidx, sem).wait()
        pltpu.sync_copy(ones, cnt_sh.at[sidx], add=True)

    plsc.subcore_barrier()
    pltpu.sync_copy(cnt_sh.at[pl.ds(ss * RSL, RSL)], zb)
    pltpu.sync_copy(zb, cntp.at[pl.ds(cc * NR + ss * RSL, RSL)])


def _sc_count(sarr):
    kern = functools.partial(
        pl.kernel,
        out_type=jax.ShapeDtypeStruct((NC * NR,), _f32),
        mesh=_sc_mesh(),
        compiler_params=pltpu.CompilerParams(use_tc_tiling_on_sc=False),
        scratch_types=[
            pltpu.VMEM_SHARED((NR,), _f32),
            pltpu.VMEM((CK,), _i32),
            pltpu.VMEM((CK,), _f32),
            pltpu.VMEM((RSL,), _f32),
            pltpu.SemaphoreType.DMA,
        ],
    )(_count_body)
    return kern(sarr)


def _make_edge_body(dh):
    def body(u_hbm, inv_hbm, g_hbm, s_hbm, d_hbm, accp,
             acc_sh, giA, siA, diA, scA, scB, rowsA, rowsB, zrow,
             su_a, su_b, ss_a, ss_b):
        cc = lax.axis_index("c")
        ss = lax.axis_index("s")
        w = cc * NS + ss

        @pl.loop(0, NSL)
        def _zero(i):
            for q in range(dh // 16):
                zrow[i, pl.ds(q * 16, 16)] = jnp.zeros((16,), _f32)

        pltpu.sync_copy(zrow, acc_sh.at[pl.ds(ss * NSL, NSL)])

        pltpu.sync_copy(g_hbm.at[pl.ds(w * CPW, CPW)], giA)
        pltpu.sync_copy(s_hbm.at[pl.ds(w * CPW, CPW)], siA)
        pltpu.sync_copy(d_hbm.at[pl.ds(w * CPW, CPW)], diA)
        plsc.subcore_barrier()

        def issue(j, rows_buf, sc_buf, sem_u, sem_s):
            pltpu.async_copy(u_hbm.at[giA.at[j]], rows_buf, sem_u)
            pltpu.async_copy(inv_hbm.at[siA.at[j]], sc_buf, sem_s)

        def drain(rows_buf, sc_buf, sem_u, sem_s):
            pltpu.make_async_copy(u_hbm.at[giA.at[0]], rows_buf, sem_u).wait()
            pltpu.make_async_copy(inv_hbm.at[siA.at[0]], sc_buf, sem_s).wait()

        def process(j, rows_buf, sc_buf, sem_u, sem_s):
            drain(rows_buf, sc_buf, sem_u, sem_s)

            @pl.loop(0, CK, unroll=4)
            def _scale(e):
                sv = sc_buf[e, pl.ds(0, 16)]
                for q in range(dh // 16):
                    v = rows_buf[e, pl.ds(q * 16, 16)]
                    rows_buf[e, pl.ds(q * 16, 16)] = v * sv

            pltpu.sync_copy(rows_buf, acc_sh.at[diA.at[j]], add=True)

        issue(0, rowsA, scA, su_a, ss_a)

        @pl.loop(0, CPW // 2)
        def _pair(i):
            j0 = 2 * i
            j1 = 2 * i + 1
            issue(j1, rowsB, scB, su_b, ss_b)
            process(j0, rowsA, scA, su_a, ss_a)
            issue(jnp.minimum(j1 + 1, CPW - 1), rowsA, scA, su_a, ss_a)
            process(j1, rowsB, scB, su_b, ss_b)

        drain(rowsA, scA, su_a, ss_a)

        plsc.subcore_barrier()
        pltpu.sync_copy(acc_sh.at[pl.ds(ss * NSL, NSL)], zrow)
        pltpu.sync_copy(zrow, accp.at[pl.ds(cc * N + ss * NSL, NSL)])
    return body


def _sc_edge_pass(u, inv, garr, sarr, darr, dh):
    kern = functools.partial(
        pl.kernel,
        out_type=pltpu.HBM((NC * N, dh), _f32),
        mesh=_sc_mesh(),
        compiler_params=pltpu.CompilerParams(use_tc_tiling_on_sc=False),
        scratch_types=[
            pltpu.VMEM_SHARED((N, dh), _f32),
            pltpu.VMEM((CPW, CK), _i32),
            pltpu.VMEM((CPW, CK), _i32),
            pltpu.VMEM((CPW, CK), _i32),
            pltpu.VMEM((CK, 16), _f32),
            pltpu.VMEM((CK, 16), _f32),
            pltpu.VMEM((CK, dh), _f32),
            pltpu.VMEM((CK, dh), _f32),
            pltpu.VMEM((NSL, dh), _f32),
            pltpu.SemaphoreType.DMA,
            pltpu.SemaphoreType.DMA,
            pltpu.SemaphoreType.DMA,
            pltpu.SemaphoreType.DMA,
        ],
    )(_make_edge_body(dh))
    return kern(u, inv, garr, sarr, darr)


def kernel(x, edge_index, edge_type, basis1, comp1, root1, bias1,
           basis2, comp2, root2, bias2):
    e2 = edge_index.reshape(2, C, CK)
    et2 = edge_type.reshape(C, CK)

    garr, sarr, darr = pl.pallas_call(
        _prep_body,
        out_shape=[
            jax.ShapeDtypeStruct((C, CK), _i32),
            jax.ShapeDtypeStruct((C, CK), _i32),
            jax.ShapeDtypeStruct((C, CK), _i32),
        ],
    )(e2, et2)

    pad_i = jnp.zeros((PADC - C, CK), _i32)
    pad_s = jnp.full((PADC - C, CK), PADSEG, _i32)
    garr_p = jnp.concatenate([garr, pad_i], axis=0)
    sarr_p = jnp.concatenate([sarr, pad_s], axis=0)
    darr_p = jnp.concatenate([darr, pad_i], axis=0)

    cntp = _sc_count(sarr)

    invrep = pl.pallas_call(
        _invrep_body,
        grid=(NR // 2000,),
        in_specs=[pl.BlockSpec((NC, 2000, 1), lambda i: (0, i, 0))],
        out_specs=pl.BlockSpec((2000, 16), lambda i: (i, 0)),
        out_shape=jax.ShapeDtypeStruct((NR, 16), _f32),
    )(cntp.reshape(NC, NR, 1))
    invrep = jnp.concatenate([invrep, jnp.zeros((16, 16), _f32)], axis=0)

    bf1 = jnp.transpose(basis1, (1, 0, 2)).reshape(D_IN, NB * DH)
    u1, xr1 = _tc_lin1(x, bf1, comp1, root1, bias1.reshape(1, DH))

    accp1 = _sc_edge_pass(u1.reshape(NR, DH), invrep, garr_p, sarr_p, darr_p, DH)

    bf2 = jnp.transpose(basis2, (1, 0, 2)).reshape(DH, NB * DO)
    u2, xr2 = _tc_mid(accp1.reshape(NC, N, DH), xr1, bf2, comp2, root2,
                      bias2.reshape(1, DO))

    accp2 = _sc_edge_pass(u2.reshape(NR, DO), invrep, garr_p, sarr_p, darr_p, DO)

    return _tc_fin(accp2.reshape(NC, N, DO), xr2)
